# Initial kernel scaffold; baseline (speedup 1.0000x reference)
import jax
import jax.numpy as jnp
from jax.experimental import pallas as pl


def kernel(x, edge_index, W_rel1, b_rel1, W_root1, W_rel2, b_rel2, W_root2, W_fc1, b_fc1, W_fc2, b_fc2):
    raise NotImplementedError("write your pallas kernel here")


# speedup vs baseline: 4.5829x; 4.5829x over previous
import functools

import jax
import jax.numpy as jnp
from jax import lax
from jax.experimental import pallas as pl
from jax.experimental.pallas import tpu as pltpu
from jax.experimental.pallas import tpu_sc as plsc

N = 10000
E = 320000
D_IN = 128
R1 = 8
R2 = 16
N1 = 32

NC = 2
NS = 16
NW = NC * NS

C = 128
CH = 80
E_PAD = NW * CH * C

N_P = 10016
N_ACC = 10240
ROWS_T = N_ACC // NS
PAD_ROW = 10008


def _tc_layer1_body(acc_ref, x_ref, wrel_ref, b_ref, wroot_ref, h1_ref):
    agg = acc_ref[0, :N_P, :] + acc_ref[1, :N_P, :]
    h1_ref[...] = jnp.maximum(
        jnp.dot(agg, wrel_ref[...], preferred_element_type=jnp.float32)
        + b_ref[...]
        + jnp.dot(x_ref[...], wroot_ref[...],
                  preferred_element_type=jnp.float32), 0.0)


def _tc_head_body(acc_ref, h1_ref, wrel_ref, b_ref, wroot_ref,
                  wfc1_ref, bfc1_ref, wfc2_ref, bfc2_ref, out_ref):
    agg = acc_ref[0, :N_P, :] + acc_ref[1, :N_P, :]
    h2 = jnp.maximum(
        jnp.dot(agg, wrel_ref[...], preferred_element_type=jnp.float32)
        + b_ref[...]
        + jnp.dot(h1_ref[...], wroot_ref[...],
                  preferred_element_type=jnp.float32), 0.0)
    h3 = jnp.maximum(
        jnp.dot(h2, wfc1_ref[...], preferred_element_type=jnp.float32)
        + bfc1_ref[...], 0.0)
    o = jnp.dot(h3, wfc2_ref[...], preferred_element_type=jnp.float32) \
        + bfc2_ref[...]
    valid = lax.broadcasted_iota(jnp.int32, (N_P, 1), 0) < N
    s = jnp.sum(jnp.where(valid, o, 0.0))
    out_ref[...] = o - s * (1.0 / N)


@functools.cache
def _make_sc_scatter_add(D):
    mesh = plsc.VectorSubcoreMesh(core_axis_name="c", subcore_axis_name="s",
                                  num_cores=NC, num_subcores=NS)

    @functools.partial(
        pl.kernel,
        out_type=jax.ShapeDtypeStruct((NC, N_ACC, D), jnp.float32),
        mesh=mesh,
        scratch_types=[
            pltpu.VMEM((CH, C), jnp.int32),
            pltpu.VMEM((CH, C), jnp.int32),
            pltpu.VMEM((C, D), jnp.float32),
            pltpu.VMEM_SHARED((N_ACC, D), jnp.float32),
        ],
        compiler_params=pltpu.CompilerParams(use_tc_tiling_on_sc=False),
    )
    def _sc_scatter_add(p_hbm, src_hbm, dst_hbm, zeros_hbm, out_hbm,
                        src_v, dst_v, rows_v, acc_sh):
        c = lax.axis_index("c")
        s = lax.axis_index("s")
        wid = s * NC + c
        row0 = s * ROWS_T

        pltpu.sync_copy(zeros_hbm.at[pl.ds(row0, ROWS_T)],
                        acc_sh.at[pl.ds(row0, ROWS_T)])

        pltpu.sync_copy(src_hbm.at[pl.ds(wid * CH, CH)], src_v)
        pltpu.sync_copy(dst_hbm.at[pl.ds(wid * CH, CH)], dst_v)
        plsc.subcore_barrier()

        def _edge_chunk(j, carry):
            pltpu.sync_copy(p_hbm.at[src_v.at[j]], rows_v)
            pltpu.sync_copy(rows_v, acc_sh.at[dst_v.at[j]], add=True)
            return carry
        lax.fori_loop(0, CH, _edge_chunk, 0)
        plsc.subcore_barrier()

        pltpu.sync_copy(acc_sh.at[pl.ds(row0, ROWS_T)],
                        out_hbm.at[c, pl.ds(row0, ROWS_T)])

    return _sc_scatter_add


def kernel(x, edge_index, W_rel1, b_rel1, W_root1, W_rel2, b_rel2, W_root2,
           W_fc1, b_fc1, W_fc2, b_fc2):
    ei = edge_index.astype(jnp.int32)
    ei = jnp.concatenate(
        [ei, jnp.full((2, E_PAD - E), PAD_ROW, jnp.int32)], axis=1)
    src2 = ei[0].reshape(NW * CH, C)
    dst2 = ei[1].reshape(NW * CH, C)

    x_pad = jnp.pad(x, ((0, N_P - N), (0, 0)))
    z128 = jnp.zeros((N_ACC, D_IN), jnp.float32)
    z8 = jnp.zeros((N_ACC, R1), jnp.float32)

    acc1 = _make_sc_scatter_add(D_IN)(x_pad, src2, dst2, z128)

    h1 = pl.pallas_call(
        _tc_layer1_body,
        out_shape=jax.ShapeDtypeStruct((N_P, R1), jnp.float32),
    )(acc1, x_pad, W_rel1, b_rel1.reshape(1, R1), W_root1)

    acc2 = _make_sc_scatter_add(R1)(h1, src2, dst2, z8)

    out = pl.pallas_call(
        _tc_head_body,
        out_shape=jax.ShapeDtypeStruct((N_P, 1), jnp.float32),
    )(acc2, h1, W_rel2, b_rel2.reshape(1, R2), W_root2,
      W_fc1, b_fc1.reshape(1, N1), W_fc2, b_fc2.reshape(1, 1))

    return out[:N]


# speedup vs baseline: 5.1516x; 1.1241x over previous
import functools

import jax
import jax.numpy as jnp
from jax import lax
from jax.experimental import pallas as pl
from jax.experimental.pallas import tpu as pltpu
from jax.experimental.pallas import tpu_sc as plsc

N = 10000
E = 320000
D_IN = 128
R1 = 8
R2 = 16
N1 = 32

NC = 2
NS = 16
NW = NC * NS

C = 128
CH = 80
E_PAD = NW * CH * C

N_P = 10016
N_ACC = 10240
ROWS_T = N_ACC // NS
PAD_ROW = 10008


def _tc_layer1_body(acc_ref, x_ref, wrel_ref, b_ref, wroot_ref, h1_ref):
    agg = acc_ref[0, :N_P, :] + acc_ref[1, :N_P, :]
    h1_ref[...] = jnp.maximum(
        jnp.dot(agg, wrel_ref[...], preferred_element_type=jnp.float32)
        + b_ref[...]
        + jnp.dot(x_ref[...], wroot_ref[...],
                  preferred_element_type=jnp.float32), 0.0)


def _tc_head_body(acc_ref, h1_ref, wrel_ref, b_ref, wroot_ref,
                  wfc1_ref, bfc1_ref, wfc2_ref, bfc2_ref, out_ref):
    agg = acc_ref[0, :N_P, :] + acc_ref[1, :N_P, :]
    h2 = jnp.maximum(
        jnp.dot(agg, wrel_ref[...], preferred_element_type=jnp.float32)
        + b_ref[...]
        + jnp.dot(h1_ref[...], wroot_ref[...],
                  preferred_element_type=jnp.float32), 0.0)
    h3 = jnp.maximum(
        jnp.dot(h2, wfc1_ref[...], preferred_element_type=jnp.float32)
        + bfc1_ref[...], 0.0)
    o = jnp.dot(h3, wfc2_ref[...], preferred_element_type=jnp.float32) \
        + bfc2_ref[...]
    valid = lax.broadcasted_iota(jnp.int32, (N_P, 1), 0) < N
    s = jnp.sum(jnp.where(valid, o, 0.0))
    out_ref[...] = o - s * (1.0 / N)


@functools.cache
def _make_sc_scatter_add(D):
    mesh = plsc.VectorSubcoreMesh(core_axis_name="c", subcore_axis_name="s",
                                  num_cores=NC, num_subcores=NS)

    @functools.partial(
        pl.kernel,
        out_type=jax.ShapeDtypeStruct((NC, N_ACC, D), jnp.float32),
        mesh=mesh,
        scratch_types=[
            pltpu.VMEM((CH // 2, C), jnp.int32),
            pltpu.VMEM((CH // 2, C), jnp.int32),
            pltpu.VMEM((C, D), jnp.float32),
            pltpu.VMEM((C, D), jnp.float32),
            pltpu.SemaphoreType.DMA,
            pltpu.SemaphoreType.DMA,
            pltpu.VMEM_SHARED((N_ACC, D), jnp.float32),
        ],
        compiler_params=pltpu.CompilerParams(use_tc_tiling_on_sc=False),
    )
    def _sc_scatter_add(p_hbm, src_hbm, dst_hbm, zeros_hbm, out_hbm,
                        src_v, dst_v, rows_a, rows_b, sem_a, sem_b, acc_sh):
        c = lax.axis_index("c")
        s = lax.axis_index("s")
        wid = s * NC + c
        row0 = s * ROWS_T

        pltpu.sync_copy(zeros_hbm.at[pl.ds(row0, ROWS_T)],
                        acc_sh.at[pl.ds(row0, ROWS_T)])

        plsc.subcore_barrier()

        CH2 = CH // 2

        def _half(h, carry):
            base = wid * CH + h * CH2
            pltpu.sync_copy(src_hbm.at[pl.ds(base, CH2)], src_v)
            pltpu.sync_copy(dst_hbm.at[pl.ds(base, CH2)], dst_v)
            pltpu.async_copy(p_hbm.at[src_v.at[0]], rows_a, sem_a)

            def _edge_pair(jj, carry2):
                j0 = jj * 2
                pltpu.async_copy(p_hbm.at[src_v.at[j0 + 1]], rows_b, sem_b)
                pltpu.make_async_copy(p_hbm.at[src_v.at[j0]], rows_a,
                                      sem_a).wait()
                pltpu.sync_copy(rows_a, acc_sh.at[dst_v.at[j0]], add=True)
                j2 = jnp.where(j0 + 2 < CH2, j0 + 2, 0)
                pltpu.async_copy(p_hbm.at[src_v.at[j2]], rows_a, sem_a)
                pltpu.make_async_copy(p_hbm.at[src_v.at[j0]], rows_b,
                                      sem_b).wait()
                pltpu.sync_copy(rows_b, acc_sh.at[dst_v.at[j0 + 1]], add=True)
                return carry2
            lax.fori_loop(0, CH2 // 2, _edge_pair, 0)
            pltpu.make_async_copy(p_hbm.at[src_v.at[0]], rows_a, sem_a).wait()
            return carry
        lax.fori_loop(0, 2, _half, 0)
        plsc.subcore_barrier()

        pltpu.sync_copy(acc_sh.at[pl.ds(row0, ROWS_T)],
                        out_hbm.at[c, pl.ds(row0, ROWS_T)])

    return _sc_scatter_add


def kernel(x, edge_index, W_rel1, b_rel1, W_root1, W_rel2, b_rel2, W_root2,
           W_fc1, b_fc1, W_fc2, b_fc2):
    ei = edge_index.astype(jnp.int32)
    ei = jnp.concatenate(
        [ei, jnp.full((2, E_PAD - E), PAD_ROW, jnp.int32)], axis=1)
    src2 = ei[0].reshape(NW * CH, C)
    dst2 = ei[1].reshape(NW * CH, C)

    x_pad = jnp.pad(x, ((0, N_P - N), (0, 0)))
    z128 = jnp.zeros((N_ACC, D_IN), jnp.float32)
    z8 = jnp.zeros((N_ACC, R1), jnp.float32)

    acc1 = _make_sc_scatter_add(D_IN)(x_pad, src2, dst2, z128)

    h1 = pl.pallas_call(
        _tc_layer1_body,
        out_shape=jax.ShapeDtypeStruct((N_P, R1), jnp.float32),
    )(acc1, x_pad, W_rel1, b_rel1.reshape(1, R1), W_root1)

    acc2 = _make_sc_scatter_add(R1)(h1, src2, dst2, z8)

    out = pl.pallas_call(
        _tc_head_body,
        out_shape=jax.ShapeDtypeStruct((N_P, 1), jnp.float32),
    )(acc2, h1, W_rel2, b_rel2.reshape(1, R2), W_root2,
      W_fc1, b_fc1.reshape(1, N1), W_fc2, b_fc2.reshape(1, 1))

    return out[:N]
